# baseline (speedup 1.0000x reference)
import functools

import jax
import jax.numpy as jnp
from jax import lax
from jax.experimental import pallas as pl
from jax.experimental.pallas import tpu as pltpu
from jax.experimental.pallas import tpu_sc as plsc

_SHAPE = (4096, 11008)
_NUMEL = _SHAPE[0] * _SHAPE[1]
_K = 1000000

_NC = 2
_NS = 16
_NW = _NC * _NS
_CH = 16384
_NCHUNK = _NUMEL // _CH
_CPW = _NCHUNK // _NW
_NBUF = 4
_LB = 1024
_KPAD = _K + 2 * _LB
_NOFF = _NCHUNK + 1
_NOFF_PAD = 2768
_NIT = (_CPW + _NBUF - 1) // _NBUF

_mesh = plsc.VectorSubcoreMesh(core_axis_name="c", subcore_axis_name="s")


@functools.partial(
    pl.kernel,
    out_type=jax.ShapeDtypeStruct((_NUMEL,), jnp.float32),
    mesh=_mesh,
    compiler_params=pltpu.CompilerParams(needs_layout_passes=False),
    scratch_types=(
        [pltpu.VMEM((_CH,), jnp.float32) for _ in range(_NBUF)]
        + [pltpu.VMEM((_LB,), jnp.int32) for _ in range(_NBUF)]
        + [pltpu.VMEM((_LB,), jnp.float32) for _ in range(_NBUF)]
        + [pltpu.VMEM((_NOFF_PAD,), jnp.int32)]
        + [pltpu.SemaphoreType.DMA for _ in range(2 * _NBUF)]
    ),
)
def _scatter_merge(tensor_hbm, idx_hbm, val_hbm, off_hbm, out_hbm, *refs):
    bufs = refs[0:_NBUF]
    idxb = refs[_NBUF:2 * _NBUF]
    valb = refs[2 * _NBUF:3 * _NBUF]
    offv = refs[3 * _NBUF]
    lsem = refs[3 * _NBUF + 1:3 * _NBUF + 1 + _NBUF]
    ssem = refs[3 * _NBUF + 1 + _NBUF:3 * _NBUF + 1 + 2 * _NBUF]

    wid = lax.axis_index("s") * _NC + lax.axis_index("c")
    pltpu.sync_copy(off_hbm, offv)

    def chunk_meta(c):
        cid = wid * _CPW + c
        base = cid * _CH
        sev = offv[pl.ds(cid, 16)]
        s = sev[0]
        e = sev[1]
        sb0 = (s // 8) * 8
        return base, s, e, sb0

    def load_descs(c, slot):
        base, _, _, sb0 = chunk_meta(c)
        return (
            pltpu.make_async_copy(
                tensor_hbm.at[pl.ds(base, _CH)], bufs[slot], lsem[slot]),
            pltpu.make_async_copy(
                idx_hbm.at[pl.ds(sb0, _LB)], idxb[slot], lsem[slot]),
            pltpu.make_async_copy(
                val_hbm.at[pl.ds(sb0, _LB)], valb[slot], lsem[slot]),
        )

    def store_desc(c, slot):
        base, _, _, _ = chunk_meta(c)
        return pltpu.make_async_copy(
            bufs[slot], out_hbm.at[pl.ds(base, _CH)], ssem[slot])

    def issue_loads(c, slot):
        for d in load_descs(c, slot):
            d.start()

    def wait_loads(c, slot):
        for d in load_descs(c, slot):
            d.wait()

    def scatter_chunk(c, slot):
        base, s, e, sb0 = chunk_meta(c)
        nseg = (e - sb0 + _LB - 1) // _LB
        ibuf = idxb[slot]
        vbuf = valb[slot]
        dbuf = bufs[slot]

        def seg_body(seg, carry):
            segstart = sb0 + seg * _LB

            @pl.when(seg >= 1)
            def _():
                pltpu.sync_copy(idx_hbm.at[pl.ds(segstart, _LB)], ibuf)
                pltpu.sync_copy(val_hbm.at[pl.ds(segstart, _LB)], vbuf)

            gs = jnp.maximum(0, (s - segstart) // 16)
            ge = (jnp.minimum(e, segstart + _LB) - segstart + 15) // 16
            ge = jnp.maximum(gs, jnp.minimum(_LB // 16, ge))

            def group_body(g, gcarry):
                go = g * 16
                pos = segstart + go + lax.iota(jnp.int32, 16)
                iv = ibuf[pl.ds(go, 16)]
                vv = vbuf[pl.ds(go, 16)]
                m = (pos >= s) & (pos < e)
                liv = jnp.where(m, iv - base, 0)
                plsc.addupdate_scatter(dbuf, [liv], vv, mask=m)
                return gcarry

            lax.fori_loop(gs, ge, group_body, 0)
            return carry

        lax.fori_loop(0, nseg, seg_body, 0)

    issue_loads(0, 0)
    issue_loads(1, 1)

    def ring_iter(it, carry):
        for b in range(_NBUF):
            c = it * _NBUF + b

            @pl.when(jnp.logical_and(c >= 2, c + 2 < _CPW))
            def _():
                store_desc(c - 2, (b - 2) % _NBUF).wait()

            @pl.when(c + 2 < _CPW)
            def _():
                issue_loads(c + 2, (b + 2) % _NBUF)

            @pl.when(c < _CPW)
            def _():
                wait_loads(c, b)
                scatter_chunk(c, b)
                store_desc(c, b).start()

        return carry

    lax.fori_loop(0, _NIT, ring_iter, 0)
    for i in range(_NBUF):
        c = _CPW - _NBUF + i
        store_desc(c, c % _NBUF).wait()


def kernel(tensor, values, indices):
    flat = tensor.reshape(-1)
    values = values.astype(jnp.float32)
    idx_p = jnp.zeros((_KPAD,), jnp.int32).at[:_K].set(indices)
    val_p = jnp.zeros((_KPAD,), jnp.float32).at[:_K].set(values)
    bounds = jnp.arange(_NOFF, dtype=jnp.int32) * _CH
    off = jnp.searchsorted(indices, bounds, side="left").astype(jnp.int32)
    off_p = jnp.zeros((_NOFF_PAD,), jnp.int32).at[:_NOFF].set(off)
    out = _scatter_merge(flat, idx_p, val_p, off_p)
    return out.reshape(_SHAPE)


# speedup vs baseline: 1.5742x; 1.5742x over previous
import functools

import jax
import jax.numpy as jnp
from jax import lax
from jax.experimental import pallas as pl
from jax.experimental.pallas import tpu as pltpu
from jax.experimental.pallas import tpu_sc as plsc

_SHAPE = (4096, 11008)
_NUMEL = _SHAPE[0] * _SHAPE[1]
_K = 1000000

_NC = 2
_NS = 16
_NW = _NC * _NS
_CH = 32768
_NCHUNK = _NUMEL // _CH
_CPW = _NCHUNK // _NW
_NBUF = 3
_NIT = (_CPW - 1) // _NBUF
_LB = 1024
_KPAD = 980 * _LB
_SENT = 2**31 - 1
_NBLK = _K // 8

_mesh = plsc.VectorSubcoreMesh(core_axis_name="c", subcore_axis_name="s")


@functools.partial(
    pl.kernel,
    out_type=jax.ShapeDtypeStruct((_NUMEL,), jnp.float32),
    mesh=_mesh,
    compiler_params=pltpu.CompilerParams(needs_layout_passes=False),
    scratch_types=(
        [pltpu.VMEM((_CH,), jnp.float32) for _ in range(_NBUF)]
        + [pltpu.VMEM((2 * _LB,), jnp.int32),
           pltpu.VMEM((2 * _LB,), jnp.float32),
           pltpu.VMEM((16,), jnp.int32)]
        + [pltpu.SemaphoreType.DMA for _ in range(2 * _NBUF + 2)]
    ),
)
def _scatter_merge(tensor_hbm, idx_hbm, val_hbm, out_hbm, *refs):
    bufs = refs[0:_NBUF]
    ib = refs[_NBUF]
    vb = refs[_NBUF + 1]
    tmp = refs[_NBUF + 2]
    lsem = refs[_NBUF + 3:2 * _NBUF + 3]
    ssem = refs[2 * _NBUF + 3:3 * _NBUF + 3]
    gsem = refs[3 * _NBUF + 3:3 * _NBUF + 5]

    wid = lax.axis_index("s") * _NC + lax.axis_index("c")

    def dense_load_desc(c, slot):
        base = (wid * _CPW + c) * _CH
        return pltpu.make_async_copy(
            tensor_hbm.at[pl.ds(base, _CH)], bufs[slot], lsem[slot])

    def dense_store_desc(c, slot):
        base = (wid * _CPW + c) * _CH
        return pltpu.make_async_copy(
            bufs[slot], out_hbm.at[pl.ds(base, _CH)], ssem[slot])

    def seg_descs(k, par):
        return (
            pltpu.make_async_copy(
                idx_hbm.at[pl.ds(k * _LB, _LB)],
                ib.at[pl.ds(par * _LB, _LB)], gsem[par]),
            pltpu.make_async_copy(
                val_hbm.at[pl.ds(k * _LB, _LB)],
                vb.at[pl.ds(par * _LB, _LB)], gsem[par]),
        )

    def issue_seg(k):
        for par in range(2):
            @pl.when(k % 2 == par)
            def _():
                for d in seg_descs(k, par):
                    d.start()

    def wait_seg(k):
        for par in range(2):
            @pl.when(k % 2 == par)
            def _():
                for d in seg_descs(k, par):
                    d.wait()

    tbase = wid * (_CPW * _CH)

    def bisect_body(_, st):
        lo, hi = st
        mid = (lo + hi) // 2
        pltpu.sync_copy(idx_hbm.at[pl.ds(mid * 8, 16)], tmp)
        v0 = tmp[pl.ds(0, 16)][0]
        ge = v0 >= tbase
        return jnp.where(ge, lo, mid), jnp.where(ge, mid, hi)

    _, blk = lax.fori_loop(0, 17, bisect_body, (jnp.int32(0), jnp.int32(_NBLK)))
    wstart = jnp.maximum(blk - 1, 0) * 8
    pltpu.sync_copy(idx_hbm.at[pl.ds(wstart, 16)], tmp)
    win = tmp[pl.ds(0, 16)]
    p0 = wstart + plsc.all_reduce_population_count(win < tbase)[0]
    k0 = p0 // _LB

    issue_seg(k0)
    issue_seg(k0 + 1)
    wait_seg(k0)

    def walk(c, slot, p, k):
        base = (wid * _CPW + c) * _CH
        chunk_end = base + _CH
        dbuf = bufs[slot]

        def cond(st):
            return jnp.logical_not(st[2])

        def body(st):
            p, k, _ = st
            kp = p // _LB

            @pl.when(kp > k)
            def _():
                wait_seg(kp)
                issue_seg(kp + 1)

            k = jnp.maximum(k, kp)
            ab = (p // 16) * 16
            go = ab % (2 * _LB)
            iv = ib[pl.ds(go, 16)]
            vv = vb[pl.ds(go, 16)]
            lane = lax.iota(jnp.int32, 16)
            m = ((ab + lane) >= p) & (iv < chunk_end)
            liv = jnp.where(m, iv - base, 0)
            plsc.addupdate_scatter(dbuf, [liv], vv, mask=m)
            nc = plsc.all_reduce_population_count(m)[0]
            done = nc < 16 - (p - ab)
            return p + nc, k, done

        p, k, _ = lax.while_loop(cond, body, (p, k, jnp.bool_(False)))
        return p, k

    dense_load_desc(0, 0).start()

    def ring_iter(it, st):
        p, k = st
        for b in range(_NBUF):
            c = it * _NBUF + b

            @pl.when(c >= 2)
            def _():
                dense_store_desc(c - 2, (b - 2) % _NBUF).wait()

            dense_load_desc(c + 1, (b + 1) % _NBUF).start()
            dense_load_desc(c, b).wait()
            p, k = walk(c, b, p, k)
            dense_store_desc(c, b).start()
        return p, k

    p, k = lax.fori_loop(0, _NIT, ring_iter, (p0, k0))

    cl = _CPW - 1
    dense_load_desc(cl, 0).wait()
    p, k = walk(cl, 0, p, k)
    dense_store_desc(cl, 0).start()
    wait_seg(k + 1)
    dense_store_desc(cl - 2, 1).wait()
    dense_store_desc(cl - 1, 2).wait()
    dense_store_desc(cl, 0).wait()


def kernel(tensor, values, indices):
    flat = tensor.reshape(-1)
    values = values.astype(jnp.float32)
    idx_p = jnp.concatenate(
        [indices, jnp.full((_KPAD - _K,), _SENT, dtype=jnp.int32)])
    val_p = jnp.concatenate([values, jnp.zeros((_KPAD - _K,), jnp.float32)])
    out = _scatter_merge(flat, idx_p, val_p)
    return out.reshape(_SHAPE)


# speedup vs baseline: 1.5943x; 1.0128x over previous
import functools

import jax
import jax.numpy as jnp
from jax import lax
from jax.experimental import pallas as pl
from jax.experimental.pallas import tpu as pltpu
from jax.experimental.pallas import tpu_sc as plsc

_SHAPE = (4096, 11008)
_NUMEL = _SHAPE[0] * _SHAPE[1]
_K = 1000000

_NC = 2
_NS = 16
_NW = _NC * _NS
_CH = 32768
_NCHUNK = _NUMEL // _CH
_CPW = _NCHUNK // _NW
_NBUF = 3
_NIT = (_CPW - 1) // _NBUF
_LB = 1024
_KSEG = _K // _LB
_KREM = _K - _KSEG * _LB
_NBLK = _K // 8

_mesh = plsc.VectorSubcoreMesh(core_axis_name="c", subcore_axis_name="s")


@functools.partial(
    pl.kernel,
    out_type=jax.ShapeDtypeStruct((_NUMEL,), jnp.float32),
    mesh=_mesh,
    compiler_params=pltpu.CompilerParams(needs_layout_passes=False),
    scratch_types=(
        [pltpu.VMEM((_CH,), jnp.float32) for _ in range(_NBUF)]
        + [pltpu.VMEM((2 * _LB,), jnp.int32),
           pltpu.VMEM((2 * _LB,), jnp.float32),
           pltpu.VMEM((16,), jnp.int32)]
        + [pltpu.SemaphoreType.DMA for _ in range(2 * _NBUF + 2)]
    ),
)
def _scatter_merge(tensor_hbm, idx_hbm, val_hbm, out_hbm, *refs):
    bufs = refs[0:_NBUF]
    ib = refs[_NBUF]
    vb = refs[_NBUF + 1]
    tmp = refs[_NBUF + 2]
    lsem = refs[_NBUF + 3:2 * _NBUF + 3]
    ssem = refs[2 * _NBUF + 3:3 * _NBUF + 3]
    gsem = refs[3 * _NBUF + 3:3 * _NBUF + 5]

    wid = lax.axis_index("s") * _NC + lax.axis_index("c")

    def dense_load_desc(c, slot):
        base = (wid * _CPW + c) * _CH
        return pltpu.make_async_copy(
            tensor_hbm.at[pl.ds(base, _CH)], bufs[slot], lsem[slot])

    def dense_store_desc(c, slot):
        base = (wid * _CPW + c) * _CH
        return pltpu.make_async_copy(
            bufs[slot], out_hbm.at[pl.ds(base, _CH)], ssem[slot])

    def seg_descs(k, par):
        return (
            pltpu.make_async_copy(
                idx_hbm.at[pl.ds(k * _LB, _LB)],
                ib.at[pl.ds(par * _LB, _LB)], gsem[par]),
            pltpu.make_async_copy(
                val_hbm.at[pl.ds(k * _LB, _LB)],
                vb.at[pl.ds(par * _LB, _LB)], gsem[par]),
        )

    _kpar = _KSEG % 2

    def seg_descs_last():
        return (
            pltpu.make_async_copy(
                idx_hbm.at[pl.ds(_KSEG * _LB, _KREM)],
                ib.at[pl.ds(_kpar * _LB, _KREM)], gsem[_kpar]),
            pltpu.make_async_copy(
                val_hbm.at[pl.ds(_KSEG * _LB, _KREM)],
                vb.at[pl.ds(_kpar * _LB, _KREM)], gsem[_kpar]),
        )

    def issue_seg(k):
        for par in range(2):
            @pl.when(jnp.logical_and(k % 2 == par, k < _KSEG))
            def _():
                for d in seg_descs(k, par):
                    d.start()

        @pl.when(k == _KSEG)
        def _():
            for d in seg_descs_last():
                d.start()

    def wait_seg(k):
        for par in range(2):
            @pl.when(jnp.logical_and(k % 2 == par, k < _KSEG))
            def _():
                for d in seg_descs(k, par):
                    d.wait()

        @pl.when(k == _KSEG)
        def _():
            for d in seg_descs_last():
                d.wait()

    tbase = wid * (_CPW * _CH)

    def bisect_body(_, st):
        lo, hi = st
        mid = (lo + hi) // 2
        pltpu.sync_copy(idx_hbm.at[pl.ds(mid * 8, 16)], tmp)
        v0 = tmp[pl.ds(0, 16)][0]
        ge = v0 >= tbase
        return jnp.where(ge, lo, mid), jnp.where(ge, mid, hi)

    _, blk = lax.fori_loop(0, 17, bisect_body,
                           (jnp.int32(0), jnp.int32(_NBLK - 1)))
    wstart = jnp.maximum(blk - 1, 0) * 8
    pltpu.sync_copy(idx_hbm.at[pl.ds(wstart, 16)], tmp)
    win = tmp[pl.ds(0, 16)]
    p0 = wstart + plsc.all_reduce_population_count(win < tbase)[0]
    k0 = p0 // _LB

    issue_seg(k0)
    issue_seg(k0 + 1)
    wait_seg(k0)

    def walk(c, slot, p, k):
        base = (wid * _CPW + c) * _CH
        chunk_end = base + _CH
        dbuf = bufs[slot]

        def cond(st):
            return jnp.logical_not(st[2])

        def body(st):
            p, k, _ = st
            kp = p // _LB

            @pl.when(kp > k)
            def _():
                wait_seg(kp)
                issue_seg(kp + 1)

            k = jnp.maximum(k, kp)
            ab = (p // 16) * 16
            go = ab % (2 * _LB)
            iv = ib[pl.ds(go, 16)]
            vv = vb[pl.ds(go, 16)]
            lane = lax.iota(jnp.int32, 16)
            pos = ab + lane
            m = (pos >= p) & (pos < _K) & (iv < chunk_end)
            liv = jnp.where(m, iv - base, 0)
            plsc.addupdate_scatter(dbuf, [liv], vv, mask=m)
            nc = plsc.all_reduce_population_count(m)[0]
            done = nc < 16 - (p - ab)
            return p + nc, k, done

        p, k, _ = lax.while_loop(cond, body, (p, k, jnp.bool_(False)))
        return p, k

    dense_load_desc(0, 0).start()

    def ring_iter(it, st):
        p, k = st
        for b in range(_NBUF):
            c = it * _NBUF + b

            @pl.when(c >= 2)
            def _():
                dense_store_desc(c - 2, (b - 2) % _NBUF).wait()

            dense_load_desc(c + 1, (b + 1) % _NBUF).start()
            dense_load_desc(c, b).wait()
            p, k = walk(c, b, p, k)
            dense_store_desc(c, b).start()
        return p, k

    p, k = lax.fori_loop(0, _NIT, ring_iter, (p0, k0))

    cl = _CPW - 1
    dense_load_desc(cl, 0).wait()
    p, k = walk(cl, 0, p, k)
    dense_store_desc(cl, 0).start()
    wait_seg(k + 1)
    dense_store_desc(cl - 2, 1).wait()
    dense_store_desc(cl - 1, 2).wait()
    dense_store_desc(cl, 0).wait()


def kernel(tensor, values, indices):
    flat = tensor.reshape(-1)
    values = values.astype(jnp.float32)
    out = _scatter_merge(flat, indices, values)
    return out.reshape(_SHAPE)
